# baseline (speedup 1.0000x reference)
import functools

import numpy as np
import jax
import jax.numpy as jnp
from jax import lax
from jax.experimental import pallas as pl
from jax.experimental.pallas import tpu as pltpu
from jax.experimental.pallas import tpu_sc as plsc

VOCAB = 1000000
D = 64
D2 = 128
MAX_LEN = 512
PAD_IDX = 0
EPS = 1e-5


def _sinusoidal_pe(max_len, d):
    pos = np.arange(max_len)[:, None].astype(np.float32)
    div = np.exp(np.arange(0, d, 2).astype(np.float32) * (-np.log(10000.0) / d))
    pe = np.zeros((max_len, d), dtype=np.float32)
    pe[:, 0::2] = np.sin(pos * div)
    pe[:, 1::2] = np.cos(pos * div)
    return pe


def _padt_body(tt_ref, out_ref):
    v = tt_ref[...]
    eye = jnp.eye(D, dtype=jnp.float32)
    t = lax.dot_general(v, eye, (((0,), (0,)), ((), ())),
                        preferred_element_type=jnp.float32)
    out_ref[...] = jnp.concatenate([t, jnp.zeros_like(t)], axis=1)


@functools.lru_cache(maxsize=None)
def _make_tc_padt(V):
    Cb = 4096
    return pl.pallas_call(
        _padt_body,
        grid=((V + Cb - 1) // Cb,),
        in_specs=[pl.BlockSpec((D, Cb), lambda i: (0, i))],
        out_specs=pl.BlockSpec((Cb, D2), lambda i: (i, 0)),
        out_shape=jax.ShapeDtypeStruct((V, D2), jnp.float32),
    )


@functools.lru_cache(maxsize=None)
def _make_sc_gather(B, L):
    n_tokens = B * L
    info = plsc.get_sparse_core_info()
    nw = info.num_cores * info.num_subcores
    per_w = n_tokens // nw
    G = 128
    n_groups = per_w // G
    NB = 5
    K = 3
    NT = 2
    n_outer = n_groups // NB
    gpl = B // G
    assert per_w % G == 0 and n_groups % NB == 0 and B % G == 0
    mesh = plsc.VectorSubcoreMesh(core_axis_name="c", subcore_axis_name="s")

    @functools.partial(
        pl.kernel,
        mesh=mesh,
        compiler_params=pltpu.CompilerParams(needs_layout_passes=False),
        out_type=jax.ShapeDtypeStruct((L, D, B), jnp.float32),
        scratch_types=[
            pltpu.VMEM((per_w,), jnp.int32),
            pltpu.VMEM((NB, G, D2), jnp.float32),
            pltpu.VMEM((NT, D, G), jnp.float32),
        ] + [pltpu.SemaphoreType.DMA] * (NB + NT),
    )
    def k(idx_hbm, table_hbm, out_hbm, idx_v, rows_v, rows_t, *sems):
        gs, ws = sems[:NB], sems[NB:]
        nc = info.num_cores
        wid = lax.axis_index("s") * nc + lax.axis_index("c")
        base = wid * per_w
        pltpu.sync_copy(idx_hbm.at[pl.ds(base, per_w)], idx_v)
        iota16 = lax.iota(jnp.int32, 16)
        zero16 = iota16 * 0
        NK = G // 16

        def fire_gather(grp, buf):
            pltpu.async_copy(
                table_hbm.at[idx_v.at[pl.ds(grp * G, G)]], rows_v.at[buf],
                gs[buf],
            )

        for b in range(K):
            fire_gather(b, b)

        def outer(o, carry):
            for b in range(NB):
                j = o * NB + b
                bf = (b + K) % NB

                @pl.when(j + K < n_groups)
                def _fire():
                    fire_gather(j + K, bf)

                pltpu.make_async_copy(
                    table_hbm.at[pl.ds(0, G)], rows_v.at[b], gs[b]
                ).wait()

                tp = j % NT
                jg = wid * n_groups + j
                l_pos = jg // gpl
                b0 = (jg % gpl) * G

                for t in range(NT):
                    @pl.when((j >= NT) & (tp == t))
                    def _drain(t=t):
                        pltpu.make_async_copy(
                            rows_t.at[t], out_hbm.at[0, :, pl.ds(0, G)], ws[t]
                        ).wait()

                for kk in range(NK):
                    tok16 = idx_v[pl.ds(j * G + kk * 16, 16)]
                    row16 = iota16 + kk * 16
                    valid = tok16 != PAD_IDX

                    @plsc.parallel_loop(0, D, unroll=16)
                    def _t(d, kk=kk, row16=row16, valid=valid):
                        v = plsc.load_gather(
                            rows_v.at[b], [row16, zero16 + d]
                        )
                        rows_t[tp, d, pl.ds(kk * 16, 16)] = jnp.where(
                            valid, v, 0.0
                        )

                for t in range(NT):
                    @pl.when(tp == t)
                    def _wb(t=t):
                        pltpu.async_copy(
                            rows_t.at[t], out_hbm.at[l_pos, :, pl.ds(b0, G)],
                            ws[t],
                        )
            return carry

        lax.fori_loop(0, n_outer, outer, 0)

        for t in range(NT):
            pltpu.make_async_copy(
                rows_t.at[t], out_hbm.at[0, :, pl.ds(0, G)], ws[t]
            ).wait()

    return k


def _ln_body(emb_ref, pe_ref, gamma_ref, beta_ref, out_ref):
    h = emb_ref[...] + pe_ref[...]
    mean = jnp.mean(h, axis=1, keepdims=True)
    c = h - mean
    var = jnp.mean(c * c, axis=1, keepdims=True)
    hn = c * lax.rsqrt(var + EPS)
    out_ref[...] = hn * gamma_ref[...] + beta_ref[...]


@functools.lru_cache(maxsize=None)
def _make_tc_ln(B, L, interpret=False):
    Lb = 8
    return pl.pallas_call(
        _ln_body,
        grid=(L // Lb,),
        in_specs=[
            pl.BlockSpec((Lb, D, B), lambda i: (i, 0, 0)),
            pl.BlockSpec((Lb, D, 1), lambda i: (i, 0, 0)),
            pl.BlockSpec((1, D, 1), lambda i: (0, 0, 0)),
            pl.BlockSpec((1, D, 1), lambda i: (0, 0, 0)),
        ],
        out_specs=pl.BlockSpec((Lb, D, B), lambda i: (i, 0, 0)),
        out_shape=jax.ShapeDtypeStruct((L, D, B), jnp.float32),
        interpret=interpret,
    )


def kernel(x, token_table, gamma, beta):
    B, L = x.shape
    ids = x.T.reshape(-1)
    table_wide = _make_tc_padt(VOCAB)(token_table.T)
    emb_t = _make_sc_gather(B, L)(ids, table_wide)
    pe_t = jnp.asarray(_sinusoidal_pe(MAX_LEN, D)[:L])[:, :, None]
    out_t = _make_tc_ln(B, L)(
        emb_t, pe_t, gamma.reshape(1, D, 1), beta.reshape(1, D, 1)
    )
    return jnp.transpose(out_t, (2, 0, 1))


# speedup vs baseline: 1.0154x; 1.0154x over previous
import functools

import numpy as np
import jax
import jax.numpy as jnp
from jax import lax
from jax.experimental import pallas as pl
from jax.experimental.pallas import tpu as pltpu
from jax.experimental.pallas import tpu_sc as plsc

VOCAB = 1000000
D = 64
D2 = 128
MAX_LEN = 512
PAD_IDX = 0
EPS = 1e-5


def _sinusoidal_pe(max_len, d):
    pos = np.arange(max_len)[:, None].astype(np.float32)
    div = np.exp(np.arange(0, d, 2).astype(np.float32) * (-np.log(10000.0) / d))
    pe = np.zeros((max_len, d), dtype=np.float32)
    pe[:, 0::2] = np.sin(pos * div)
    pe[:, 1::2] = np.cos(pos * div)
    return pe


def _padt_body(tt_ref, out_ref):
    v = tt_ref[...]
    t = jnp.transpose(v, (1, 0))
    out_ref[...] = jnp.concatenate([t, jnp.zeros_like(t)], axis=1)


@functools.lru_cache(maxsize=None)
def _make_tc_padt(V):
    Cb = 4096
    return pl.pallas_call(
        _padt_body,
        grid=((V + Cb - 1) // Cb,),
        in_specs=[pl.BlockSpec((D, Cb), lambda i: (0, i))],
        out_specs=pl.BlockSpec((Cb, D2), lambda i: (i, 0)),
        out_shape=jax.ShapeDtypeStruct((V, D2), jnp.float32),
    )


@functools.lru_cache(maxsize=None)
def _make_sc_gather(B, L):
    n_tokens = B * L
    info = plsc.get_sparse_core_info()
    nw = info.num_cores * info.num_subcores
    per_w = n_tokens // nw
    G = 128
    n_groups = per_w // G
    NB = 5
    K = 4
    NT = 2
    n_outer = n_groups // NB
    gpl = B // G
    assert per_w % G == 0 and n_groups % NB == 0 and B % G == 0
    mesh = plsc.VectorSubcoreMesh(core_axis_name="c", subcore_axis_name="s")

    @functools.partial(
        pl.kernel,
        mesh=mesh,
        compiler_params=pltpu.CompilerParams(needs_layout_passes=False),
        out_type=jax.ShapeDtypeStruct((L, D, B), jnp.float32),
        scratch_types=[
            pltpu.VMEM((per_w,), jnp.int32),
            pltpu.VMEM((NB, G, D2), jnp.float32),
            pltpu.VMEM((NT, D, G), jnp.float32),
        ] + [pltpu.SemaphoreType.DMA] * (NB + NT),
    )
    def k(idx_hbm, table_hbm, out_hbm, idx_v, rows_v, rows_t, *sems):
        gs, ws = sems[:NB], sems[NB:]
        nc = info.num_cores
        wid = lax.axis_index("s") * nc + lax.axis_index("c")
        base = wid * per_w
        pltpu.sync_copy(idx_hbm.at[pl.ds(base, per_w)], idx_v)
        iota16 = lax.iota(jnp.int32, 16)
        zero16 = iota16 * 0
        NK = G // 16

        def fire_gather(grp, buf):
            pltpu.async_copy(
                table_hbm.at[idx_v.at[pl.ds(grp * G, G)]], rows_v.at[buf],
                gs[buf],
            )

        for b in range(K):
            fire_gather(b, b)

        def outer(o, carry):
            for b in range(NB):
                j = o * NB + b
                bf = (b + K) % NB

                @pl.when(j + K < n_groups)
                def _fire():
                    fire_gather(j + K, bf)

                pltpu.make_async_copy(
                    table_hbm.at[pl.ds(0, G)], rows_v.at[b], gs[b]
                ).wait()

                tp = j % NT
                jg = wid * n_groups + j
                l_pos = jg // gpl
                b0 = (jg % gpl) * G

                for t in range(NT):
                    @pl.when((j >= NT) & (tp == t))
                    def _drain(t=t):
                        pltpu.make_async_copy(
                            rows_t.at[t], out_hbm.at[0, :, pl.ds(0, G)], ws[t]
                        ).wait()

                for kk in range(NK):
                    tok16 = idx_v[pl.ds(j * G + kk * 16, 16)]
                    row16 = iota16 + kk * 16
                    valid = tok16 != PAD_IDX

                    @plsc.parallel_loop(0, D, unroll=16)
                    def _t(d, kk=kk, row16=row16, valid=valid):
                        v = plsc.load_gather(
                            rows_v.at[b], [row16, zero16 + d]
                        )
                        rows_t[tp, d, pl.ds(kk * 16, 16)] = jnp.where(
                            valid, v, 0.0
                        )

                for t in range(NT):
                    @pl.when(tp == t)
                    def _wb(t=t):
                        pltpu.async_copy(
                            rows_t.at[t], out_hbm.at[l_pos, :, pl.ds(b0, G)],
                            ws[t],
                        )
            return carry

        lax.fori_loop(0, n_outer, outer, 0)

        for t in range(NT):
            pltpu.make_async_copy(
                rows_t.at[t], out_hbm.at[0, :, pl.ds(0, G)], ws[t]
            ).wait()

    return k


def _ln_body(emb_ref, pe_ref, gamma_ref, beta_ref, out_ref):
    h = emb_ref[...] + pe_ref[...]
    mean = jnp.mean(h, axis=1, keepdims=True)
    c = h - mean
    var = jnp.mean(c * c, axis=1, keepdims=True)
    hn = c * lax.rsqrt(var + EPS)
    out_ref[...] = hn * gamma_ref[...] + beta_ref[...]


@functools.lru_cache(maxsize=None)
def _make_tc_ln(B, L, interpret=False):
    Lb = 8
    return pl.pallas_call(
        _ln_body,
        grid=(L // Lb,),
        in_specs=[
            pl.BlockSpec((Lb, D, B), lambda i: (i, 0, 0)),
            pl.BlockSpec((Lb, D, 1), lambda i: (i, 0, 0)),
            pl.BlockSpec((1, D, 1), lambda i: (0, 0, 0)),
            pl.BlockSpec((1, D, 1), lambda i: (0, 0, 0)),
        ],
        out_specs=pl.BlockSpec((Lb, D, B), lambda i: (i, 0, 0)),
        out_shape=jax.ShapeDtypeStruct((L, D, B), jnp.float32),
        interpret=interpret,
    )


def kernel(x, token_table, gamma, beta):
    B, L = x.shape
    ids = x.T.reshape(-1)
    table_wide = _make_tc_padt(VOCAB)(token_table.T)
    emb_t = _make_sc_gather(B, L)(ids, table_wide)
    pe_t = jnp.asarray(_sinusoidal_pe(MAX_LEN, D)[:L])[:, :, None]
    out_t = _make_tc_ln(B, L)(
        emb_t, pe_t, gamma.reshape(1, D, 1), beta.reshape(1, D, 1)
    )
    return jnp.transpose(out_t, (2, 0, 1))
